# baseline (speedup 1.0000x reference)
import functools

import jax
import jax.numpy as jnp
from jax import lax
from jax.experimental import pallas as pl
from jax.experimental.pallas import tpu as pltpu
from jax.experimental.pallas import tpu_sc as plsc

EMB = 64
CHUNK = 128


@functools.cache
def _make_gather(n_rows: int):
    info = plsc.get_sparse_core_info()
    nc, ns = info.num_cores, info.num_subcores
    nw = nc * ns
    n_chunks = n_rows // CHUNK
    per_w = n_chunks // nw
    mesh = plsc.VectorSubcoreMesh(core_axis_name="c", subcore_axis_name="s")

    @functools.partial(
        pl.kernel,
        out_type=jax.ShapeDtypeStruct((n_rows, EMB), jnp.float32),
        mesh=mesh,
        scratch_types=[
            pltpu.VMEM((per_w, CHUNK), jnp.int32),
            pltpu.VMEM((CHUNK, EMB), jnp.float32),
            pltpu.SemaphoreType.DMA,
        ],
        compiler_params=pltpu.CompilerParams(use_tc_tiling_on_sc=False),
    )
    def gather_kernel(idx_hbm, table_hbm, out_hbm, idx_v, rows_v, sem):
        wid = lax.axis_index("s") * nc + lax.axis_index("c")
        base = wid * per_w
        pltpu.sync_copy(idx_hbm.at[pl.ds(base, per_w)], idx_v)

        @pl.loop(0, per_w)
        def _(j):
            pltpu.async_copy(table_hbm.at[idx_v.at[j]], rows_v, sem).wait()
            pltpu.sync_copy(rows_v, out_hbm.at[pl.ds((base + j) * CHUNK, CHUNK)])

    return gather_kernel


def kernel(inputs, table):
    b, t = inputs.shape
    n = b * t
    idx = inputs.reshape(n // CHUNK, CHUNK).astype(jnp.int32)
    out = _make_gather(n)(idx, table)
    return out.reshape(b, t, EMB)


# speedup vs baseline: 1.1064x; 1.1064x over previous
import functools

import jax
import jax.numpy as jnp
from jax import lax
from jax.experimental import pallas as pl
from jax.experimental.pallas import tpu as pltpu
from jax.experimental.pallas import tpu_sc as plsc

EMB = 64
CHUNK = 128
R = 4


@functools.cache
def _make_gather(n_rows: int):
    info = plsc.get_sparse_core_info()
    nc, ns = info.num_cores, info.num_subcores
    nw = nc * ns
    n_chunks = n_rows // CHUNK
    per_w = n_chunks // nw
    laps = per_w // R
    assert per_w % (2 * R) == 0
    mesh = plsc.VectorSubcoreMesh(core_axis_name="c", subcore_axis_name="s")

    @functools.partial(
        pl.kernel,
        out_type=jax.ShapeDtypeStruct((n_rows, EMB), jnp.float32),
        mesh=mesh,
        scratch_types=[
            pltpu.VMEM((per_w, CHUNK), jnp.int32),
            pltpu.VMEM((R, CHUNK, EMB), jnp.float32),
            pltpu.VMEM((R, CHUNK, EMB), jnp.float32),
            pltpu.SemaphoreType.DMA,
            pltpu.SemaphoreType.DMA,
            pltpu.SemaphoreType.DMA,
            pltpu.SemaphoreType.DMA,
        ],
        compiler_params=pltpu.CompilerParams(use_tc_tiling_on_sc=False),
    )
    def gather_kernel(idx_hbm, table_hbm, out_hbm, idx_v, bufa, bufb,
                      gsema, gsemb, ssema, ssemb):
        wid = lax.axis_index("s") * nc + lax.axis_index("c")
        c0 = wid * per_w
        pltpu.sync_copy(idx_hbm.at[pl.ds(c0, per_w)], idx_v)

        def fire_gathers(arr, sem, j0):
            for b in range(R):
                pltpu.async_copy(table_hbm.at[idx_v.at[j0 + b]], arr.at[b], sem)

        def fire_stores(arr, sem, j0):
            for b in range(R):
                pltpu.async_copy(
                    arr.at[b], out_hbm.at[pl.ds((c0 + j0 + b) * CHUNK, CHUNK)], sem)

        def drain(arr, sem):
            for b in range(R):
                pltpu.make_async_copy(
                    table_hbm.at[pl.ds(0, CHUNK)], arr.at[b], sem).wait()

        def drain_stores(arr, sem):
            for b in range(R):
                pltpu.make_async_copy(
                    arr.at[b], out_hbm.at[pl.ds(b * CHUNK, CHUNK)], sem).wait()

        fire_gathers(bufa, gsema, 0)

        @pl.loop(0, laps, step=2)
        def _(g):
            j0 = g * R

            @pl.when(g > 0)
            def _():
                drain_stores(bufb, ssemb)
            fire_gathers(bufb, gsemb, j0 + R)
            drain(bufa, gsema)
            fire_stores(bufa, ssema, j0)
            drain(bufb, gsemb)
            fire_stores(bufb, ssemb, j0 + R)

            @pl.when(g + 2 < laps)
            def _():
                drain_stores(bufa, ssema)
                fire_gathers(bufa, gsema, j0 + 2 * R)

        drain_stores(bufa, ssema)
        drain_stores(bufb, ssemb)

    return gather_kernel


def kernel(inputs, table):
    b, t = inputs.shape
    n = b * t
    idx = inputs.reshape(n // CHUNK, CHUNK).astype(jnp.int32)
    out = _make_gather(n)(idx, table)
    return out.reshape(b, t, EMB)
